# baseline (speedup 1.0000x reference)
import jax
import jax.numpy as jnp
from jax.experimental import pallas as pl
from jax.experimental.pallas import tpu as pltpu

_NUM_EMB = 8192
_HALF = _NUM_EMB // 2
_DIM = 32
_ROW_BLOCK = 256


def _vq_body(x_ref, emb_ref, idx_ref, q_ref, loss_ref):
    i = pl.program_id(0)
    x = x_ref[...]
    emb = emb_ref[...]
    isq = jnp.sum(x * x, axis=1, keepdims=True)
    esq = jnp.sum(emb * emb, axis=1)
    mm = jax.lax.dot_general(
        x, emb, dimension_numbers=(((1,), (1,)), ((), ())),
        preferred_element_type=jnp.float32)
    d = isq - 2.0 * mm + esq[None, :]

    d_lo = d[:, :_HALF]
    d_hi = d[:, _HALF:]
    v_lo = jnp.min(d_lo, axis=1, keepdims=True)
    v_hi = jnp.min(d_hi, axis=1, keepdims=True)
    am_lo = jnp.argmin(d_lo, axis=1).astype(jnp.int32)
    am_hi = jnp.argmin(d_hi, axis=1).astype(jnp.int32)

    key_lo = v_lo.astype(jnp.bfloat16).astype(jnp.float32)
    key_hi = jax.lax.bitcast_convert_type(
        jax.lax.bitcast_convert_type(v_hi, jnp.uint32)
        & jnp.uint32(0xFFFF8000), jnp.float32)
    pick_lo = (key_lo <= key_hi)[:, 0]
    am = jnp.where(pick_lo, am_lo, am_hi + _HALF)
    idx_ref[...] = am[:, None]

    onehot = (jax.lax.broadcasted_iota(jnp.int32, d.shape, 1)
              == am[:, None]).astype(jnp.float32)
    q = jax.lax.dot_general(
        onehot, emb, dimension_numbers=(((1,), (0,)), ((), ())),
        precision=jax.lax.Precision.HIGHEST,
        preferred_element_type=jnp.float32)
    q_ref[...] = q

    @pl.when(i == 0)
    def _init():
        loss_ref[...] = jnp.zeros((1, 1), jnp.float32)

    loss_ref[...] += jnp.sum((q - x) ** 2).reshape(1, 1)


def kernel(inputs, embeddings):
    in_shape = inputs.shape
    rows = in_shape[0] * in_shape[1]
    flat_x = inputs.reshape(rows, _DIM)
    grid = rows // _ROW_BLOCK
    idx, quantized, loss_sum = pl.pallas_call(
        _vq_body,
        grid=(grid,),
        in_specs=[
            pl.BlockSpec((_ROW_BLOCK, _DIM), lambda i: (i, 0)),
            pl.BlockSpec((_NUM_EMB, _DIM), lambda i: (0, 0)),
        ],
        out_specs=[
            pl.BlockSpec((_ROW_BLOCK, 1), lambda i: (i, 0)),
            pl.BlockSpec((_ROW_BLOCK, _DIM), lambda i: (i, 0)),
            pl.BlockSpec((1, 1), lambda i: (0, 0)),
        ],
        out_shape=[
            jax.ShapeDtypeStruct((rows, 1), jnp.int32),
            jax.ShapeDtypeStruct((rows, _DIM), jnp.float32),
            jax.ShapeDtypeStruct((1, 1), jnp.float32),
        ],
    )(flat_x, embeddings)
    quantized = quantized.reshape(in_shape)
    loss = loss_sum[0, 0] / jnp.float32(rows * _DIM)
    encoding_indices = idx.reshape(in_shape[:-1])
    return (quantized, loss, encoding_indices)


# speedup vs baseline: 3.0666x; 3.0666x over previous
import jax
import jax.numpy as jnp
from jax.experimental import pallas as pl
from jax.experimental.pallas import tpu as pltpu
from jax.experimental.pallas import tpu_sc as plsc

_NUM_EMB = 8192
_HALF = _NUM_EMB // 2
_DIM = 32
_ROW_BLOCK = 512
_GATHER_WINDOW = 128


def _vq_body(x_ref, emb_ref, idx_ref, loss_ref):
    i = pl.program_id(0)
    x = x_ref[...]
    emb = emb_ref[...]
    isq = jnp.sum(x * x, axis=1, keepdims=True)
    esq = jnp.sum(emb * emb, axis=1)
    mm = jax.lax.dot_general(
        x, emb, dimension_numbers=(((1,), (1,)), ((), ())),
        preferred_element_type=jnp.float32)
    d = isq - 2.0 * mm + esq[None, :]

    d_lo = d[:, :_HALF]
    d_hi = d[:, _HALF:]
    v_lo = jnp.min(d_lo, axis=1, keepdims=True)
    v_hi = jnp.min(d_hi, axis=1, keepdims=True)
    am_lo = jnp.argmin(d_lo, axis=1).astype(jnp.int32)
    am_hi = jnp.argmin(d_hi, axis=1).astype(jnp.int32)

    key_lo = v_lo.astype(jnp.bfloat16).astype(jnp.float32)
    key_hi = jax.lax.bitcast_convert_type(
        jax.lax.bitcast_convert_type(v_hi, jnp.uint32)
        & jnp.uint32(0xFFFF8000), jnp.float32)
    pick_lo = key_lo <= key_hi
    am = jnp.where(pick_lo[:, 0], am_lo, am_hi + _HALF)
    idx_ref[...] = am[:, None]

    picked_v = jnp.where(pick_lo, v_lo, v_hi)

    @pl.when(i == 0)
    def _init():
        loss_ref[...] = jnp.zeros((1, 1), jnp.float32)

    loss_ref[...] += jnp.sum(picked_v).reshape(1, 1)


def _sc_gather(emb_padded, idx_row):
    rows = idx_row.shape[1]
    width = emb_padded.shape[1]
    mesh = plsc.VectorSubcoreMesh(core_axis_name="core",
                                  subcore_axis_name="subcore")

    @pl.kernel(out_type=jax.ShapeDtypeStruct((rows, width), emb_padded.dtype),
               mesh=mesh)
    def gather_kernel(x_hbm, i_hbm, o_hbm):
        def body(i_vmem, o_vmem):
            pltpu.sync_copy(x_hbm.at[i_vmem.at[0]], o_vmem)

        pltpu.emit_pipeline(
            body,
            grid=(rows // _GATHER_WINDOW,),
            in_specs=[pl.BlockSpec((1, _GATHER_WINDOW),
                                   index_map=lambda i: (0, i))],
            out_specs=[pl.BlockSpec((_GATHER_WINDOW, width),
                                    index_map=lambda i: (i, 0))],
            core_axis_name="subcore",
            dimension_semantics=(pltpu.PARALLEL,),
        )(i_hbm, o_hbm)

    return gather_kernel(emb_padded, idx_row)


def kernel(inputs, embeddings):
    in_shape = inputs.shape
    rows = in_shape[0] * in_shape[1]
    flat_x = inputs.reshape(rows, _DIM)
    grid = rows // _ROW_BLOCK
    idx, loss_sum = pl.pallas_call(
        _vq_body,
        grid=(grid,),
        in_specs=[
            pl.BlockSpec((_ROW_BLOCK, _DIM), lambda i: (i, 0)),
            pl.BlockSpec((_NUM_EMB, _DIM), lambda i: (0, 0)),
        ],
        out_specs=[
            pl.BlockSpec((_ROW_BLOCK, 1), lambda i: (i, 0)),
            pl.BlockSpec((1, 1), lambda i: (0, 0)),
        ],
        out_shape=[
            jax.ShapeDtypeStruct((rows, 1), jnp.int32),
            jax.ShapeDtypeStruct((1, 1), jnp.float32),
        ],
    )(flat_x, embeddings)
    emb_padded = jnp.pad(embeddings, ((0, 0), (0, 128 - _DIM)))
    gathered = _sc_gather(emb_padded, idx.reshape(1, rows))
    quantized = gathered[:, :_DIM].reshape(in_shape)
    loss = loss_sum[0, 0] / jnp.float32(rows * _DIM)
    encoding_indices = idx.reshape(in_shape[:-1])
    return (quantized, loss, encoding_indices)
